# baseline (speedup 1.0000x reference)
import functools

import jax
import jax.numpy as jnp
from jax import lax
from jax.experimental import pallas as pl
from jax.experimental.pallas import tpu as pltpu
from jax.experimental.pallas import tpu_sc as plsc

VECDIM = 32
HIST_PAD = 256
NBUF = 8


@functools.cache
def _build_gather(bsz: int, hist: int, D: int, nbuf: int):
    info = plsc.get_sparse_core_info()
    nc, ns = info.num_cores, info.num_subcores
    nw = nc * ns
    assert bsz % nw == 0
    r_per_w = bsz // nw
    assert r_per_w % nbuf == 0
    n_groups = r_per_w // nbuf
    mesh = plsc.VectorSubcoreMesh(core_axis_name="c", subcore_axis_name="s")

    @functools.partial(
        pl.kernel,
        mesh=mesh,
        out_type=jax.ShapeDtypeStruct((bsz * hist, D), jnp.float32),
        scratch_types=[
            pltpu.VMEM((r_per_w, HIST_PAD), jnp.int32),
            [pltpu.VMEM((hist, D), jnp.float32) for _ in range(nbuf)],
            [pltpu.SemaphoreType.DMA for _ in range(nbuf)],
            [pltpu.SemaphoreType.DMA for _ in range(nbuf)],
        ],
        compiler_params=pltpu.CompilerParams(use_tc_tiling_on_sc=False),
    )
    def k(xp_hbm, table_hbm, out_hbm, idx2d, rows, sg, so):
        wid = lax.axis_index("s") * nc + lax.axis_index("c")
        row0 = wid * r_per_w
        base = row0 * hist

        pltpu.sync_copy(xp_hbm.at[pl.ds(row0, r_per_w), :], idx2d)

        def start_gather(r, b):
            pltpu.async_copy(
                table_hbm.at[idx2d.at[r, pl.ds(0, hist)]], rows[b], sg[b])

        def start_write(r, b):
            pltpu.async_copy(
                rows[b], out_hbm.at[pl.ds(base + r * hist, hist)], so[b])

        def drain(sem, b):
            pltpu.make_async_copy(
                table_hbm.at[pl.ds(0, hist)], rows[b], sem).wait()

        for b in range(nbuf):
            start_gather(b, b)

        def body(g, carry):
            for b in range(nbuf):
                drain(sg[b], b)
                start_write(g * nbuf + b, b)

            @pl.when(g + 1 < n_groups)
            def _():
                for b in range(nbuf):
                    drain(so[b], b)
                    start_gather((g + 1) * nbuf + b, b)

            return carry

        lax.fori_loop(0, n_groups, body, 0)
        for b in range(nbuf):
            drain(so[b], b)

    return k


def _expand_body(x_ref, o_ref):
    o_ref[...] = x_ref[...].reshape(o_ref.shape)


@functools.cache
def _build_expand(bsz: int, hist: int, D: int, rows_per_blk: int):
    grid = bsz // rows_per_blk
    blk = rows_per_blk * hist
    return pl.pallas_call(
        _expand_body,
        grid=(grid,),
        in_specs=[pl.BlockSpec((blk, D), lambda i: (i, 0))],
        out_specs=pl.BlockSpec((rows_per_blk, hist, D), lambda i: (i, 0, 0)),
        out_shape=jax.ShapeDtypeStruct((bsz, hist, D), jnp.float32),
    )


def kernel(x, table):
    bsz, hist = x.shape
    xp = jnp.pad(x, ((0, 0), (0, HIST_PAD - hist)))
    out2d = _build_gather(bsz, hist, VECDIM, NBUF)(xp, table)
    return _build_expand(bsz, hist, VECDIM, 128)(out2d)


# speedup vs baseline: 1.0207x; 1.0207x over previous
import functools

import jax
import jax.numpy as jnp
from jax import lax
from jax.experimental import pallas as pl
from jax.experimental.pallas import tpu as pltpu
from jax.experimental.pallas import tpu_sc as plsc

VECDIM = 32


@functools.cache
def _build_gather(B: int, D: int, chunk: int):
    info = plsc.get_sparse_core_info()
    nc, ns = info.num_cores, info.num_subcores
    nw = nc * ns
    assert B % nw == 0
    b_per_w = B // nw
    assert b_per_w % chunk == 0
    n_chunks = b_per_w // chunk
    mesh = plsc.VectorSubcoreMesh(core_axis_name="c", subcore_axis_name="s")

    @functools.partial(
        pl.kernel,
        mesh=mesh,
        out_type=jax.ShapeDtypeStruct((B, D), jnp.float32),
        scratch_types=[
            pltpu.VMEM((b_per_w,), jnp.int32),
            pltpu.VMEM((chunk, D), jnp.float32),
            pltpu.VMEM((chunk, D), jnp.float32),
            pltpu.SemaphoreType.DMA,
            pltpu.SemaphoreType.DMA,
            pltpu.SemaphoreType.DMA,
            pltpu.SemaphoreType.DMA,
        ],
        compiler_params=pltpu.CompilerParams(use_tc_tiling_on_sc=False),
    )
    def k(idx_hbm, table_hbm, out_hbm, idx_all, rows0, rows1,
          sg0, sg1, so0, so1):
        wid = lax.axis_index("s") * nc + lax.axis_index("c")
        base = wid * b_per_w
        rows = (rows0, rows1)
        sg = (sg0, sg1)
        so = (so0, so1)

        pltpu.sync_copy(idx_hbm.at[pl.ds(base, b_per_w)], idx_all)

        def gather_chunk(c, b):
            return pltpu.async_copy(
                table_hbm.at[idx_all.at[pl.ds(c * chunk, chunk)]],
                rows[b], sg[b])

        gather = [None, None]
        outcp = [None, None]
        gather[0] = gather_chunk(0, 0)
        for c in range(n_chunks):
            b = c % 2
            gather[b].wait()
            outcp[b] = pltpu.async_copy(
                rows[b], out_hbm.at[pl.ds(base + c * chunk, chunk)], so[b])
            if c + 1 < n_chunks:
                if outcp[1 - b] is not None:
                    outcp[1 - b].wait()
                gather[1 - b] = gather_chunk(c + 1, 1 - b)
        outcp[(n_chunks - 1) % 2].wait()
        if n_chunks > 1:
            outcp[n_chunks % 2].wait()

    return k


def kernel(x, table):
    bsz, hist = x.shape
    B = bsz * hist
    one_i = lax.optimization_barrier(jnp.int32(1))
    one_f = lax.optimization_barrier(jnp.float32(1.0))
    one_t = lax.optimization_barrier(jnp.float32(1.0))
    x_flat = (x * one_i).reshape(B)
    t128 = lax.optimization_barrier((table * one_t).reshape(250000, 128))
    tlin = t128.reshape(1000000, VECDIM)
    out2d = _build_gather(B, VECDIM, 1600)(x_flat, tlin)
    return out2d.reshape(bsz, hist, VECDIM) * one_f


# speedup vs baseline: 1.0239x; 1.0031x over previous
import functools

import jax
import jax.numpy as jnp
from jax import lax
from jax.experimental import pallas as pl
from jax.experimental.pallas import tpu as pltpu
from jax.experimental.pallas import tpu_sc as plsc

VECDIM = 32


@functools.cache
def _build_gather(B: int, D: int, chunk: int):
    info = plsc.get_sparse_core_info()
    nc, ns = info.num_cores, info.num_subcores
    nw = nc * ns
    assert B % nw == 0
    b_per_w = B // nw
    assert b_per_w % chunk == 0
    n_chunks = b_per_w // chunk
    mesh = plsc.VectorSubcoreMesh(core_axis_name="c", subcore_axis_name="s")

    @functools.partial(
        pl.kernel,
        mesh=mesh,
        out_type=jax.ShapeDtypeStruct((B, D), jnp.float32),
        scratch_types=[
            pltpu.VMEM((b_per_w,), jnp.int32),
            pltpu.VMEM((chunk, D), jnp.float32),
            pltpu.SemaphoreType.DMA,
        ],
        compiler_params=pltpu.CompilerParams(use_tc_tiling_on_sc=False),
    )
    def k(idx_hbm, table_hbm, out_hbm, idx_all, rows, sg):
        wid = lax.axis_index("s") * nc + lax.axis_index("c")
        base = wid * b_per_w

        pltpu.sync_copy(idx_hbm.at[pl.ds(base, b_per_w)], idx_all)

        for c in range(n_chunks):
            pltpu.async_copy(
                table_hbm.at[idx_all.at[pl.ds(c * chunk, chunk)]],
                rows, sg).wait()
            pltpu.sync_copy(rows, out_hbm.at[pl.ds(base + c * chunk, chunk)])

    return k


def kernel(x, table):
    bsz, hist = x.shape
    B = bsz * hist
    one_i = lax.optimization_barrier(jnp.int32(1))
    one_f = lax.optimization_barrier(jnp.float32(1.0))
    one_t = lax.optimization_barrier(jnp.float32(1.0))
    x_flat = (x * one_i).reshape(B)
    t128 = lax.optimization_barrier((table * one_t).reshape(250000, 128))
    tlin = t128.reshape(1000000, VECDIM)
    out2d = _build_gather(B, VECDIM, 3200)(x_flat, tlin)
    return out2d.reshape(bsz, hist, VECDIM) * one_f


# speedup vs baseline: 1.3776x; 1.3454x over previous
import functools

import jax
import jax.numpy as jnp
from jax import lax
from jax.experimental import pallas as pl
from jax.experimental.pallas import tpu as pltpu
from jax.experimental.pallas import tpu_sc as plsc

VECDIM = 32


@functools.cache
def _build_gather(B: int, D: int, chunk: int):
    info = plsc.get_sparse_core_info()
    nc, ns = info.num_cores, info.num_subcores
    nw = nc * ns
    assert B % nw == 0
    b_per_w = B // nw
    assert b_per_w % chunk == 0
    n_chunks = b_per_w // chunk
    mesh = plsc.VectorSubcoreMesh(core_axis_name="c", subcore_axis_name="s")

    @functools.partial(
        pl.kernel,
        mesh=mesh,
        out_type=jax.ShapeDtypeStruct((B, D), jnp.float32),
        scratch_types=[
            pltpu.VMEM((b_per_w,), jnp.int32),
            pltpu.VMEM((chunk, D), jnp.float32),
            pltpu.SemaphoreType.DMA,
        ],
        compiler_params=pltpu.CompilerParams(use_tc_tiling_on_sc=False),
    )
    def k(idx_hbm, table_hbm, out_hbm, idx_all, rows, sg):
        wid = lax.axis_index("s") * nc + lax.axis_index("c")
        base = wid * b_per_w

        pltpu.sync_copy(idx_hbm.at[pl.ds(base, b_per_w)], idx_all)

        for c in range(n_chunks):
            pltpu.async_copy(
                table_hbm.at[idx_all.at[pl.ds(c * chunk, chunk)]],
                rows, sg).wait()
            pltpu.sync_copy(rows, out_hbm.at[pl.ds(base + c * chunk, chunk)])

    return k


def kernel(x, table):
    bsz, hist = x.shape
    B = bsz * hist
    out2d = _build_gather(B, VECDIM, 3200)(x.reshape(B), table)
    return out2d.reshape(bsz, hist, VECDIM)


# speedup vs baseline: 1.3792x; 1.0012x over previous
import functools

import jax
import jax.numpy as jnp
from jax import lax
from jax.experimental import pallas as pl
from jax.experimental.pallas import tpu as pltpu
from jax.experimental.pallas import tpu_sc as plsc

VECDIM = 32


@functools.cache
def _build_gather(B: int, D: int, chunk: int):
    info = plsc.get_sparse_core_info()
    nc, ns = info.num_cores, info.num_subcores
    nw = nc * ns
    assert B % nw == 0
    b_per_w = B // nw
    assert b_per_w % chunk == 0
    n_chunks = b_per_w // chunk
    mesh = plsc.VectorSubcoreMesh(core_axis_name="c", subcore_axis_name="s")

    @functools.partial(
        pl.kernel,
        mesh=mesh,
        out_type=jax.ShapeDtypeStruct((B, D), jnp.float32),
        scratch_types=[
            pltpu.VMEM((b_per_w,), jnp.int32),
            pltpu.VMEM((4, chunk, D), jnp.float32),
            pltpu.SemaphoreType.DMA,
            pltpu.SemaphoreType.DMA,
            pltpu.SemaphoreType.DMA,
            pltpu.SemaphoreType.DMA,
        ],
        compiler_params=pltpu.CompilerParams(use_tc_tiling_on_sc=False),
    )
    def k(idx_hbm, table_hbm, out_hbm, idx_all, rows, s0, s1, s2, s3):
        wid = lax.axis_index("s") * nc + lax.axis_index("c")
        base = wid * b_per_w
        K = 4
        sems = (s0, s1, s2, s3)

        pltpu.sync_copy(idx_hbm.at[pl.ds(base, b_per_w)], idx_all)

        def gather_chunk(c, b):
            return pltpu.async_copy(
                table_hbm.at[idx_all.at[pl.ds(c * chunk, chunk)]],
                rows.at[b], sems[b])

        g = [None] * K
        for b in range(min(K, n_chunks)):
            g[b] = gather_chunk(b, b)
        for c in range(n_chunks):
            b = c % K
            g[b].wait()
            pltpu.sync_copy(rows.at[b], out_hbm.at[pl.ds(base + c * chunk, chunk)])
            if c + K < n_chunks:
                g[b] = gather_chunk(c + K, b)

    return k


def kernel(x, table):
    bsz, hist = x.shape
    B = bsz * hist
    out2d = _build_gather(B, VECDIM, 800)(x.reshape(B), table)
    return out2d.reshape(bsz, hist, VECDIM)
